# baseline (speedup 1.0000x reference)
import functools

import jax
import jax.numpy as jnp
from jax import lax
from jax.experimental import pallas as pl
from jax.experimental.pallas import tpu as pltpu
from jax.experimental.pallas import tpu_sc as plsc

N_NODES = 10000
F = 128
NPAD = 10240
PAD_NODE = N_NODES
DEG_W = 128
CHUNK = 128
GRP = 2
NC = 2
NS = 16
NW = NC * NS
ROWS_PER_TILE = NPAD // NS

_mesh = plsc.VectorSubcoreMesh(core_axis_name="c", subcore_axis_name="s")


def _sc_degree(edges_il, ones_hbm, zeros_hbm, chunks_per_worker):
    cpw = chunks_per_worker
    ngroups = cpw // GRP

    @functools.partial(
        pl.kernel,
        out_type=jax.ShapeDtypeStruct((NC * NPAD, DEG_W), jnp.float32),
        mesh=_mesh,
        scratch_types=[
            pltpu.VMEM((GRP, 2, CHUNK), jnp.int32),
            pltpu.VMEM((GRP, 2, CHUNK), jnp.int32),
            pltpu.VMEM((CHUNK, DEG_W), jnp.float32),
            pltpu.VMEM_SHARED((NPAD, DEG_W), jnp.float32),
            pltpu.SemaphoreType.DMA,
            pltpu.SemaphoreType.DMA,
            pltpu.SemaphoreType.DMA,
        ],
    )
    def deg_kernel(e_hbm, ones_ref, zeros_ref, out_hbm, eb0, eb1, ones_v,
                   acc_sh, isem, sem0, sem1):
        c = lax.axis_index("c")
        s = lax.axis_index("s")
        r0 = s * ROWS_PER_TILE
        pltpu.sync_copy(zeros_ref.at[pl.ds(r0, ROWS_PER_TILE)],
                        acc_sh.at[pl.ds(r0, ROWS_PER_TILE)])
        pltpu.sync_copy(ones_ref, ones_v)
        plsc.subcore_barrier()
        wid = c * NS + s
        gbase = wid * cpw
        sems = (sem0, sem1)

        pltpu.async_copy(e_hbm.at[pl.ds(gbase, GRP)], eb0, isem)

        @pl.loop(0, ngroups, step=2)
        def _(g):
            for p, (eb, ebn) in ((0, (eb0, eb1)), (1, (eb1, eb0))):
                gg = g + p
                pltpu.make_async_copy(
                    e_hbm.at[pl.ds(gbase + gg * GRP, GRP)], eb, isem).wait()

                @pl.when(gg + 1 < ngroups)
                def _():
                    pltpu.async_copy(
                        e_hbm.at[pl.ds(gbase + (gg + 1) * GRP, GRP)], ebn, isem)

                hs = [pltpu.async_copy(ones_v, acc_sh.at[eb.at[k, 1]],
                                       sems[k], add=True) for k in range(GRP)]
                for h in hs:
                    h.wait()

        plsc.subcore_barrier()
        pltpu.sync_copy(acc_sh.at[pl.ds(r0, ROWS_PER_TILE)],
                        out_hbm.at[pl.ds(c * NPAD + r0, ROWS_PER_TILE)])

    return deg_kernel(edges_il, ones_hbm, zeros_hbm)


def _sc_aggregate(y, edges_il, zeros_hbm, chunks_per_worker):
    cpw = chunks_per_worker
    ngroups = cpw // GRP

    @functools.partial(
        pl.kernel,
        out_type=jax.ShapeDtypeStruct((NC * NPAD, F), jnp.float32),
        mesh=_mesh,
        scratch_types=[
            pltpu.VMEM((GRP, 2, CHUNK), jnp.int32),
            pltpu.VMEM((GRP, 2, CHUNK), jnp.int32),
            pltpu.VMEM((CHUNK, F), jnp.float32),
            pltpu.VMEM((CHUNK, F), jnp.float32),
            pltpu.VMEM_SHARED((NPAD, F), jnp.float32),
            pltpu.SemaphoreType.DMA,
            pltpu.SemaphoreType.DMA,
            pltpu.SemaphoreType.DMA,
        ],
    )
    def agg_kernel(y_hbm, e_hbm, zeros_ref, out_hbm,
                   eb0, eb1, buf0, buf1, acc_sh,
                   isem, sem0, sem1):
        c = lax.axis_index("c")
        s = lax.axis_index("s")
        r0 = s * ROWS_PER_TILE
        pltpu.sync_copy(zeros_ref.at[pl.ds(r0, ROWS_PER_TILE)],
                        acc_sh.at[pl.ds(r0, ROWS_PER_TILE)])
        plsc.subcore_barrier()
        wid = c * NS + s
        gbase = wid * cpw
        bufs = (buf0, buf1)
        sems = (sem0, sem1)

        pltpu.async_copy(e_hbm.at[pl.ds(gbase, GRP)], eb0, isem)

        @pl.loop(0, ngroups, step=2)
        def _(g):
            for p, (eb, ebn) in ((0, (eb0, eb1)), (1, (eb1, eb0))):
                gg = g + p
                pltpu.make_async_copy(
                    e_hbm.at[pl.ds(gbase + gg * GRP, GRP)], eb, isem).wait()

                @pl.when(gg + 1 < ngroups)
                def _():
                    pltpu.async_copy(
                        e_hbm.at[pl.ds(gbase + (gg + 1) * GRP, GRP)], ebn, isem)

                hs = [pltpu.async_copy(y_hbm.at[eb.at[k, 0]], bufs[k], sems[k])
                      for k in range(GRP)]
                for k in range(GRP):
                    hs[k].wait()
                    pltpu.sync_copy(bufs[k], acc_sh.at[eb.at[k, 1]], add=True)

        plsc.subcore_barrier()
        pltpu.sync_copy(acc_sh.at[pl.ds(r0, ROWS_PER_TILE)],
                        out_hbm.at[pl.ds(c * NPAD + r0, ROWS_PER_TILE)])

    return agg_kernel(y, edges_il, zeros_hbm)


R = 1024
GRID = NPAD // R


def _dinv_of(deg_blk):
    return lax.rsqrt(deg_blk[0, :, 0:1] + deg_blk[1, :, 0:1] + 1.0)


def _tc_first(x_pad, W1, degs):
    def body(x_ref, w_ref, deg_ref, y_ref):
        dinv = _dinv_of(deg_ref[...])
        y = lax.dot_general(x_ref[...], w_ref[...], (((1,), (0,)), ((), ())),
                            precision=lax.Precision.HIGHEST,
                            preferred_element_type=jnp.float32)
        y_ref[...] = y * dinv

    return pl.pallas_call(
        body,
        grid=(GRID,),
        in_specs=[
            pl.BlockSpec((R, F), lambda i: (i, 0)),
            pl.BlockSpec((F, F), lambda i: (0, 0)),
            pl.BlockSpec((NC, R, DEG_W), lambda i: (0, i, 0)),
        ],
        out_specs=pl.BlockSpec((R, F), lambda i: (i, 0)),
        out_shape=jax.ShapeDtypeStruct((NPAD, F), jnp.float32),
    )(x_pad, W1, degs)


def _tc_middle(agg1, y1, degs, W2, b1):
    def body(a_ref, y1_ref, deg_ref, w_ref, b_ref, y2_ref):
        dinv = _dinv_of(deg_ref[...])
        a = a_ref[...]
        pre = dinv * (a[0] + a[1] + y1_ref[...]) + b_ref[...]
        h = jnp.maximum(pre, 0.0)
        y2 = lax.dot_general(h, w_ref[...], (((1,), (0,)), ((), ())),
                             precision=lax.Precision.HIGHEST,
                             preferred_element_type=jnp.float32)
        y2_ref[...] = y2 * dinv

    return pl.pallas_call(
        body,
        grid=(GRID,),
        in_specs=[
            pl.BlockSpec((NC, R, F), lambda i: (0, i, 0)),
            pl.BlockSpec((R, F), lambda i: (i, 0)),
            pl.BlockSpec((NC, R, DEG_W), lambda i: (0, i, 0)),
            pl.BlockSpec((F, F), lambda i: (0, 0)),
            pl.BlockSpec((1, F), lambda i: (0, 0)),
        ],
        out_specs=pl.BlockSpec((R, F), lambda i: (i, 0)),
        out_shape=jax.ShapeDtypeStruct((NPAD, F), jnp.float32),
    )(agg1, y1, degs, W2, b1)


def _tc_last(agg2, y2, degs, b2):
    def body(a_ref, y2_ref, deg_ref, b_ref, o_ref):
        dinv = _dinv_of(deg_ref[...])
        a = a_ref[...]
        pre = dinv * (a[0] + a[1] + y2_ref[...]) + b_ref[...]
        o_ref[...] = jax.nn.sigmoid(pre)

    return pl.pallas_call(
        body,
        grid=(GRID,),
        in_specs=[
            pl.BlockSpec((NC, R, F), lambda i: (0, i, 0)),
            pl.BlockSpec((R, F), lambda i: (i, 0)),
            pl.BlockSpec((NC, R, DEG_W), lambda i: (0, i, 0)),
            pl.BlockSpec((1, F), lambda i: (0, 0)),
        ],
        out_specs=pl.BlockSpec((R, F), lambda i: (i, 0)),
        out_shape=jax.ShapeDtypeStruct((NPAD, F), jnp.float32),
    )(agg2, y2, degs, b2)


def kernel(x, edge_index, W1, b1, W2, b2):
    src = edge_index[0].astype(jnp.int32)
    dst = edge_index[1].astype(jnp.int32)
    n_edges = src.shape[0]
    total_chunks = -(-n_edges // CHUNK)
    total_chunks = -(-total_chunks // (2 * GRP * NW)) * (2 * GRP * NW)
    e_pad = total_chunks * CHUNK - n_edges
    cpw = total_chunks // NW
    pad = jnp.full((e_pad,), PAD_NODE, jnp.int32)
    src_pad = jnp.concatenate([src, pad]).reshape(total_chunks, CHUNK)
    dst_pad = jnp.concatenate([dst, pad]).reshape(total_chunks, CHUNK)
    edges_il = jnp.stack([src_pad, dst_pad], axis=1)

    x_pad = jnp.pad(x, ((0, NPAD - x.shape[0]), (0, 0)))
    ones_hbm = jnp.ones((CHUNK, DEG_W), jnp.float32)
    zeros_deg = jnp.zeros((NPAD, DEG_W), jnp.float32)
    zeros_f = jnp.zeros((NPAD, F), jnp.float32)

    degs = _sc_degree(edges_il, ones_hbm, zeros_deg, cpw).reshape(NC, NPAD, DEG_W)
    y1 = _tc_first(x_pad, W1, degs)
    agg1 = _sc_aggregate(y1, edges_il, zeros_f, cpw).reshape(NC, NPAD, F)
    y2 = _tc_middle(agg1, y1, degs, W2, b1.reshape(1, F))
    agg2 = _sc_aggregate(y2, edges_il, zeros_f, cpw).reshape(NC, NPAD, F)
    out = _tc_last(agg2, y2, degs, b2.reshape(1, F))
    return out[:N_NODES]


# speedup vs baseline: 1.2430x; 1.2430x over previous
import functools

import jax
import jax.numpy as jnp
from jax import lax
from jax.experimental import pallas as pl
from jax.experimental.pallas import tpu as pltpu
from jax.experimental.pallas import tpu_sc as plsc

N_NODES = 10000
F = 128
NPAD = 10240
PAD_NODE = N_NODES
DEG_W = 128
CHUNK = 128
NC = 2
NS = 16
NW = NC * NS
ROWS_PER_TILE = NPAD // NS

_mesh = plsc.VectorSubcoreMesh(core_axis_name="c", subcore_axis_name="s")


def _sc_degree(dst_pad, ones_hbm, zeros_hbm, chunks_per_worker):
    @functools.partial(
        pl.kernel,
        out_type=jax.ShapeDtypeStruct((NC * NPAD, DEG_W), jnp.float32),
        mesh=_mesh,
        scratch_types=[
            pltpu.VMEM((CHUNK,), jnp.int32),
            pltpu.VMEM((CHUNK, DEG_W), jnp.float32),
            pltpu.VMEM_SHARED((NPAD, DEG_W), jnp.float32),
        ],
    )
    def deg_kernel(dst_hbm, ones_ref, zeros_ref, out_hbm, dst_v, ones_v, acc_sh):
        c = lax.axis_index("c")
        s = lax.axis_index("s")
        r0 = s * ROWS_PER_TILE
        pltpu.sync_copy(zeros_ref.at[pl.ds(r0, ROWS_PER_TILE)],
                        acc_sh.at[pl.ds(r0, ROWS_PER_TILE)])
        pltpu.sync_copy(ones_ref, ones_v)
        plsc.subcore_barrier()
        wid = c * NS + s

        @pl.loop(0, chunks_per_worker)
        def _(i):
            base = (wid * chunks_per_worker + i) * CHUNK
            pltpu.sync_copy(dst_hbm.at[pl.ds(base, CHUNK)], dst_v)
            pltpu.sync_copy(ones_v, acc_sh.at[dst_v], add=True)

        plsc.subcore_barrier()
        pltpu.sync_copy(acc_sh.at[pl.ds(r0, ROWS_PER_TILE)],
                        out_hbm.at[pl.ds(c * NPAD + r0, ROWS_PER_TILE)])

    return deg_kernel(dst_pad, ones_hbm, zeros_hbm)


def _sc_aggregate(y, src_pad, dst_pad, zeros_hbm, chunks_per_worker):
    @functools.partial(
        pl.kernel,
        out_type=jax.ShapeDtypeStruct((NC * NPAD, F), jnp.float32),
        mesh=_mesh,
        scratch_types=[
            pltpu.VMEM((CHUNK,), jnp.int32),
            pltpu.VMEM((CHUNK,), jnp.int32),
            pltpu.VMEM((CHUNK, F), jnp.float32),
            pltpu.VMEM_SHARED((NPAD, F), jnp.float32),
            pltpu.SemaphoreType.DMA,
        ],
    )
    def agg_kernel(y_hbm, src_hbm, dst_hbm, zeros_ref, out_hbm,
                   src_v, dst_v, buf_v, acc_sh, sem):
        c = lax.axis_index("c")
        s = lax.axis_index("s")
        r0 = s * ROWS_PER_TILE
        pltpu.sync_copy(zeros_ref.at[pl.ds(r0, ROWS_PER_TILE)],
                        acc_sh.at[pl.ds(r0, ROWS_PER_TILE)])
        plsc.subcore_barrier()
        wid = c * NS + s

        @pl.loop(0, chunks_per_worker)
        def _(i):
            base = (wid * chunks_per_worker + i) * CHUNK
            pltpu.sync_copy(src_hbm.at[pl.ds(base, CHUNK)], src_v)
            pltpu.sync_copy(dst_hbm.at[pl.ds(base, CHUNK)], dst_v)
            pltpu.async_copy(y_hbm.at[src_v], buf_v, sem).wait()
            pltpu.sync_copy(buf_v, acc_sh.at[dst_v], add=True)

        plsc.subcore_barrier()
        pltpu.sync_copy(acc_sh.at[pl.ds(r0, ROWS_PER_TILE)],
                        out_hbm.at[pl.ds(c * NPAD + r0, ROWS_PER_TILE)])

    return agg_kernel(y, src_pad, dst_pad, zeros_hbm)


R = 1024
GRID = NPAD // R


def _dinv_of(deg_blk):
    return lax.rsqrt(deg_blk[0, :, 0:1] + deg_blk[1, :, 0:1] + 1.0)


def _tc_first(x_pad, W1, degs):
    def body(x_ref, w_ref, deg_ref, y_ref):
        dinv = _dinv_of(deg_ref[...])
        y = lax.dot_general(x_ref[...], w_ref[...], (((1,), (0,)), ((), ())),
                            precision=lax.Precision.HIGHEST,
                            preferred_element_type=jnp.float32)
        y_ref[...] = y * dinv

    return pl.pallas_call(
        body,
        grid=(GRID,),
        in_specs=[
            pl.BlockSpec((R, F), lambda i: (i, 0)),
            pl.BlockSpec((F, F), lambda i: (0, 0)),
            pl.BlockSpec((NC, R, DEG_W), lambda i: (0, i, 0)),
        ],
        out_specs=pl.BlockSpec((R, F), lambda i: (i, 0)),
        out_shape=jax.ShapeDtypeStruct((NPAD, F), jnp.float32),
    )(x_pad, W1, degs)


def _tc_middle(agg1, y1, degs, W2, b1):
    def body(a_ref, y1_ref, deg_ref, w_ref, b_ref, y2_ref):
        dinv = _dinv_of(deg_ref[...])
        a = a_ref[...]
        pre = dinv * (a[0] + a[1] + y1_ref[...]) + b_ref[...]
        h = jnp.maximum(pre, 0.0)
        y2 = lax.dot_general(h, w_ref[...], (((1,), (0,)), ((), ())),
                             precision=lax.Precision.HIGHEST,
                             preferred_element_type=jnp.float32)
        y2_ref[...] = y2 * dinv

    return pl.pallas_call(
        body,
        grid=(GRID,),
        in_specs=[
            pl.BlockSpec((NC, R, F), lambda i: (0, i, 0)),
            pl.BlockSpec((R, F), lambda i: (i, 0)),
            pl.BlockSpec((NC, R, DEG_W), lambda i: (0, i, 0)),
            pl.BlockSpec((F, F), lambda i: (0, 0)),
            pl.BlockSpec((1, F), lambda i: (0, 0)),
        ],
        out_specs=pl.BlockSpec((R, F), lambda i: (i, 0)),
        out_shape=jax.ShapeDtypeStruct((NPAD, F), jnp.float32),
    )(agg1, y1, degs, W2, b1)


def _tc_last(agg2, y2, degs, b2):
    def body(a_ref, y2_ref, deg_ref, b_ref, o_ref):
        dinv = _dinv_of(deg_ref[...])
        a = a_ref[...]
        pre = dinv * (a[0] + a[1] + y2_ref[...]) + b_ref[...]
        o_ref[...] = jax.nn.sigmoid(pre)

    return pl.pallas_call(
        body,
        grid=(GRID,),
        in_specs=[
            pl.BlockSpec((NC, R, F), lambda i: (0, i, 0)),
            pl.BlockSpec((R, F), lambda i: (i, 0)),
            pl.BlockSpec((NC, R, DEG_W), lambda i: (0, i, 0)),
            pl.BlockSpec((1, F), lambda i: (0, 0)),
        ],
        out_specs=pl.BlockSpec((R, F), lambda i: (i, 0)),
        out_shape=jax.ShapeDtypeStruct((NPAD, F), jnp.float32),
    )(agg2, y2, degs, b2)


def kernel(x, edge_index, W1, b1, W2, b2):
    src = edge_index[0].astype(jnp.int32)
    dst = edge_index[1].astype(jnp.int32)
    n_edges = src.shape[0]
    total_chunks = -(-n_edges // CHUNK)
    total_chunks = -(-total_chunks // NW) * NW
    e_pad = total_chunks * CHUNK - n_edges
    cpw = total_chunks // NW
    src_pad = jnp.concatenate([src, jnp.full((e_pad,), PAD_NODE, jnp.int32)])
    dst_pad = jnp.concatenate([dst, jnp.full((e_pad,), PAD_NODE, jnp.int32)])

    x_pad = jnp.pad(x, ((0, NPAD - x.shape[0]), (0, 0)))
    ones_hbm = jnp.ones((CHUNK, DEG_W), jnp.float32)
    zeros_deg = jnp.zeros((NPAD, DEG_W), jnp.float32)
    zeros_f = jnp.zeros((NPAD, F), jnp.float32)

    degs = _sc_degree(dst_pad, ones_hbm, zeros_deg, cpw).reshape(NC, NPAD, DEG_W)
    y1 = _tc_first(x_pad, W1, degs)
    agg1 = _sc_aggregate(y1, src_pad, dst_pad, zeros_f, cpw).reshape(NC, NPAD, F)
    y2 = _tc_middle(agg1, y1, degs, W2, b1.reshape(1, F))
    agg2 = _sc_aggregate(y2, src_pad, dst_pad, zeros_f, cpw).reshape(NC, NPAD, F)
    out = _tc_last(agg2, y2, degs, b2.reshape(1, F))
    return out[:N_NODES]


# speedup vs baseline: 1.2660x; 1.0185x over previous
import functools

import jax
import jax.numpy as jnp
from jax import lax
from jax.experimental import pallas as pl
from jax.experimental.pallas import tpu as pltpu
from jax.experimental.pallas import tpu_sc as plsc

N_NODES = 10000
F = 128
NPAD = 10240
PAD_NODE = N_NODES
DEG_W = 128
CHUNK = 128
NC = 2
NS = 16
NW = NC * NS
ROWS_PER_TILE = NPAD // NS

_mesh = plsc.VectorSubcoreMesh(core_axis_name="c", subcore_axis_name="s")


def _sc_degree(dst_pad, ones_hbm, zeros_hbm, chunks_per_worker):
    @functools.partial(
        pl.kernel,
        out_type=jax.ShapeDtypeStruct((NC * NPAD, DEG_W), jnp.float32),
        mesh=_mesh,
        scratch_types=[
            pltpu.VMEM((CHUNK,), jnp.int32),
            pltpu.VMEM((CHUNK,), jnp.int32),
            pltpu.VMEM((CHUNK, DEG_W), jnp.float32),
            pltpu.VMEM_SHARED((NPAD, DEG_W), jnp.float32),
            pltpu.SemaphoreType.DMA,
            pltpu.SemaphoreType.DMA,
        ],
    )
    def deg_kernel(dst_hbm, ones_ref, zeros_ref, out_hbm, dst_va, dst_vb,
                   ones_v, acc_sh, sema, semb):
        c = lax.axis_index("c")
        s = lax.axis_index("s")
        r0 = s * ROWS_PER_TILE
        pltpu.sync_copy(zeros_ref.at[pl.ds(r0, ROWS_PER_TILE)],
                        acc_sh.at[pl.ds(r0, ROWS_PER_TILE)])
        pltpu.sync_copy(ones_ref, ones_v)
        plsc.subcore_barrier()
        wid = c * NS + s

        @pl.loop(0, chunks_per_worker // 2)
        def _(j):
            base = (wid * chunks_per_worker + 2 * j) * CHUNK
            pltpu.sync_copy(dst_hbm.at[pl.ds(base, CHUNK)], dst_va)
            ha = pltpu.async_copy(ones_v, acc_sh.at[dst_va], sema, add=True)
            pltpu.sync_copy(dst_hbm.at[pl.ds(base + CHUNK, CHUNK)], dst_vb)
            hb = pltpu.async_copy(ones_v, acc_sh.at[dst_vb], semb, add=True)
            ha.wait()
            hb.wait()

        if chunks_per_worker % 2:
            tail = (wid * chunks_per_worker + chunks_per_worker - 1) * CHUNK
            pltpu.sync_copy(dst_hbm.at[pl.ds(tail, CHUNK)], dst_va)
            pltpu.sync_copy(ones_v, acc_sh.at[dst_va], add=True)

        plsc.subcore_barrier()
        pltpu.sync_copy(acc_sh.at[pl.ds(r0, ROWS_PER_TILE)],
                        out_hbm.at[pl.ds(c * NPAD + r0, ROWS_PER_TILE)])

    return deg_kernel(dst_pad, ones_hbm, zeros_hbm)


def _sc_aggregate(y, src_pad, dst_pad, zeros_hbm, chunks_per_worker):
    @functools.partial(
        pl.kernel,
        out_type=jax.ShapeDtypeStruct((NC * NPAD, F), jnp.float32),
        mesh=_mesh,
        scratch_types=[
            pltpu.VMEM((CHUNK,), jnp.int32),
            pltpu.VMEM((CHUNK,), jnp.int32),
            pltpu.VMEM((CHUNK, F), jnp.float32),
            pltpu.VMEM_SHARED((NPAD, F), jnp.float32),
            pltpu.SemaphoreType.DMA,
        ],
    )
    def agg_kernel(y_hbm, src_hbm, dst_hbm, zeros_ref, out_hbm,
                   src_v, dst_v, buf_v, acc_sh, sem):
        c = lax.axis_index("c")
        s = lax.axis_index("s")
        r0 = s * ROWS_PER_TILE
        pltpu.sync_copy(zeros_ref.at[pl.ds(r0, ROWS_PER_TILE)],
                        acc_sh.at[pl.ds(r0, ROWS_PER_TILE)])
        plsc.subcore_barrier()
        wid = c * NS + s

        @pl.loop(0, chunks_per_worker)
        def _(i):
            base = (wid * chunks_per_worker + i) * CHUNK
            pltpu.sync_copy(src_hbm.at[pl.ds(base, CHUNK)], src_v)
            pltpu.sync_copy(dst_hbm.at[pl.ds(base, CHUNK)], dst_v)
            pltpu.async_copy(y_hbm.at[src_v], buf_v, sem).wait()
            pltpu.sync_copy(buf_v, acc_sh.at[dst_v], add=True)

        plsc.subcore_barrier()
        pltpu.sync_copy(acc_sh.at[pl.ds(r0, ROWS_PER_TILE)],
                        out_hbm.at[pl.ds(c * NPAD + r0, ROWS_PER_TILE)])

    return agg_kernel(y, src_pad, dst_pad, zeros_hbm)


R = 1024
GRID = NPAD // R


def _dinv_of(deg_blk):
    return lax.rsqrt(deg_blk[0, :, 0:1] + deg_blk[1, :, 0:1] + 1.0)


def _tc_first(x_pad, W1, degs):
    def body(x_ref, w_ref, deg_ref, y_ref):
        dinv = _dinv_of(deg_ref[...])
        y = lax.dot_general(x_ref[...], w_ref[...], (((1,), (0,)), ((), ())),
                            precision=lax.Precision.HIGHEST,
                            preferred_element_type=jnp.float32)
        y_ref[...] = y * dinv

    return pl.pallas_call(
        body,
        grid=(GRID,),
        in_specs=[
            pl.BlockSpec((R, F), lambda i: (i, 0)),
            pl.BlockSpec((F, F), lambda i: (0, 0)),
            pl.BlockSpec((NC, R, DEG_W), lambda i: (0, i, 0)),
        ],
        out_specs=pl.BlockSpec((R, F), lambda i: (i, 0)),
        out_shape=jax.ShapeDtypeStruct((NPAD, F), jnp.float32),
    )(x_pad, W1, degs)


def _tc_middle(agg1, y1, degs, W2, b1):
    def body(a_ref, y1_ref, deg_ref, w_ref, b_ref, y2_ref):
        dinv = _dinv_of(deg_ref[...])
        a = a_ref[...]
        pre = dinv * (a[0] + a[1] + y1_ref[...]) + b_ref[...]
        h = jnp.maximum(pre, 0.0)
        y2 = lax.dot_general(h, w_ref[...], (((1,), (0,)), ((), ())),
                             precision=lax.Precision.HIGHEST,
                             preferred_element_type=jnp.float32)
        y2_ref[...] = y2 * dinv

    return pl.pallas_call(
        body,
        grid=(GRID,),
        in_specs=[
            pl.BlockSpec((NC, R, F), lambda i: (0, i, 0)),
            pl.BlockSpec((R, F), lambda i: (i, 0)),
            pl.BlockSpec((NC, R, DEG_W), lambda i: (0, i, 0)),
            pl.BlockSpec((F, F), lambda i: (0, 0)),
            pl.BlockSpec((1, F), lambda i: (0, 0)),
        ],
        out_specs=pl.BlockSpec((R, F), lambda i: (i, 0)),
        out_shape=jax.ShapeDtypeStruct((NPAD, F), jnp.float32),
    )(agg1, y1, degs, W2, b1)


def _tc_last(agg2, y2, degs, b2):
    def body(a_ref, y2_ref, deg_ref, b_ref, o_ref):
        dinv = _dinv_of(deg_ref[...])
        a = a_ref[...]
        pre = dinv * (a[0] + a[1] + y2_ref[...]) + b_ref[...]
        o_ref[...] = jax.nn.sigmoid(pre)

    return pl.pallas_call(
        body,
        grid=(GRID,),
        in_specs=[
            pl.BlockSpec((NC, R, F), lambda i: (0, i, 0)),
            pl.BlockSpec((R, F), lambda i: (i, 0)),
            pl.BlockSpec((NC, R, DEG_W), lambda i: (0, i, 0)),
            pl.BlockSpec((1, F), lambda i: (0, 0)),
        ],
        out_specs=pl.BlockSpec((R, F), lambda i: (i, 0)),
        out_shape=jax.ShapeDtypeStruct((NPAD, F), jnp.float32),
    )(agg2, y2, degs, b2)


def kernel(x, edge_index, W1, b1, W2, b2):
    src = edge_index[0].astype(jnp.int32)
    dst = edge_index[1].astype(jnp.int32)
    n_edges = src.shape[0]
    total_chunks = -(-n_edges // CHUNK)
    total_chunks = -(-total_chunks // NW) * NW
    e_pad = total_chunks * CHUNK - n_edges
    cpw = total_chunks // NW
    src_pad = jnp.concatenate([src, jnp.full((e_pad,), PAD_NODE, jnp.int32)])
    dst_pad = jnp.concatenate([dst, jnp.full((e_pad,), PAD_NODE, jnp.int32)])

    x_pad = jnp.pad(x, ((0, NPAD - x.shape[0]), (0, 0)))
    ones_hbm = jnp.ones((CHUNK, DEG_W), jnp.float32)
    zeros_deg = jnp.zeros((NPAD, DEG_W), jnp.float32)
    zeros_f = jnp.zeros((NPAD, F), jnp.float32)

    degs = _sc_degree(dst_pad, ones_hbm, zeros_deg, cpw).reshape(NC, NPAD, DEG_W)
    y1 = _tc_first(x_pad, W1, degs)
    agg1 = _sc_aggregate(y1, src_pad, dst_pad, zeros_f, cpw).reshape(NC, NPAD, F)
    y2 = _tc_middle(agg1, y1, degs, W2, b1.reshape(1, F))
    agg2 = _sc_aggregate(y2, src_pad, dst_pad, zeros_f, cpw).reshape(NC, NPAD, F)
    out = _tc_last(agg2, y2, degs, b2.reshape(1, F))
    return out[:N_NODES]


# speedup vs baseline: 1.3366x; 1.0558x over previous
import functools

import jax
import jax.numpy as jnp
from jax import lax
from jax.experimental import pallas as pl
from jax.experimental.pallas import tpu as pltpu
from jax.experimental.pallas import tpu_sc as plsc

N_NODES = 10000
F = 128
NPAD = 10240
PAD_NODE = N_NODES
DEG_W = 128
CHUNK = 128
NC = 2
NS = 16
NW = NC * NS
ROWS_PER_TILE = NPAD // NS

_mesh = plsc.VectorSubcoreMesh(core_axis_name="c", subcore_axis_name="s")


def _sc_degree(dst_pad, ones_hbm, zeros_hbm, chunks_per_worker):
    @functools.partial(
        pl.kernel,
        out_type=jax.ShapeDtypeStruct((NC * NPAD, DEG_W), jnp.float32),
        mesh=_mesh,
        scratch_types=[
            pltpu.VMEM((CHUNK,), jnp.int32),
            pltpu.VMEM((CHUNK,), jnp.int32),
            pltpu.VMEM((CHUNK, DEG_W), jnp.float32),
            pltpu.VMEM_SHARED((NPAD, DEG_W), jnp.float32),
            pltpu.SemaphoreType.DMA,
            pltpu.SemaphoreType.DMA,
        ],
    )
    def deg_kernel(dst_hbm, ones_ref, zeros_ref, out_hbm, dst_va, dst_vb,
                   ones_v, acc_sh, sema, semb):
        c = lax.axis_index("c")
        s = lax.axis_index("s")
        r0 = s * ROWS_PER_TILE
        pltpu.sync_copy(zeros_ref.at[pl.ds(r0, ROWS_PER_TILE)],
                        acc_sh.at[pl.ds(r0, ROWS_PER_TILE)])
        pltpu.sync_copy(ones_ref, ones_v)
        plsc.subcore_barrier()
        wid = c * NS + s

        @pl.loop(0, chunks_per_worker // 2)
        def _(j):
            base = (wid * chunks_per_worker + 2 * j) * CHUNK
            pltpu.sync_copy(dst_hbm.at[pl.ds(base, CHUNK)], dst_va)
            ha = pltpu.async_copy(ones_v, acc_sh.at[dst_va], sema, add=True)
            pltpu.sync_copy(dst_hbm.at[pl.ds(base + CHUNK, CHUNK)], dst_vb)
            hb = pltpu.async_copy(ones_v, acc_sh.at[dst_vb], semb, add=True)
            ha.wait()
            hb.wait()

        if chunks_per_worker % 2:
            tail = (wid * chunks_per_worker + chunks_per_worker - 1) * CHUNK
            pltpu.sync_copy(dst_hbm.at[pl.ds(tail, CHUNK)], dst_va)
            pltpu.sync_copy(ones_v, acc_sh.at[dst_va], add=True)

        plsc.subcore_barrier()
        pltpu.sync_copy(acc_sh.at[pl.ds(r0, ROWS_PER_TILE)],
                        out_hbm.at[pl.ds(c * NPAD + r0, ROWS_PER_TILE)])

    return deg_kernel(dst_pad, ones_hbm, zeros_hbm)


def _sc_aggregate(y, src_pad, dst_pad, zeros_hbm, chunks_per_worker):
    @functools.partial(
        pl.kernel,
        out_type=jax.ShapeDtypeStruct((NC * NPAD, F), jnp.float32),
        mesh=_mesh,
        scratch_types=[
            pltpu.VMEM((CHUNK,), jnp.int32),
            pltpu.VMEM((CHUNK,), jnp.int32),
            pltpu.VMEM((CHUNK,), jnp.int32),
            pltpu.VMEM((CHUNK,), jnp.int32),
            pltpu.VMEM((CHUNK, F), jnp.float32),
            pltpu.VMEM((CHUNK, F), jnp.float32),
            pltpu.VMEM_SHARED((NPAD, F), jnp.float32),
            pltpu.SemaphoreType.DMA,
            pltpu.SemaphoreType.DMA,
            pltpu.SemaphoreType.DMA,
            pltpu.SemaphoreType.DMA,
        ],
    )
    def agg_kernel(y_hbm, src_hbm, dst_hbm, zeros_ref, out_hbm,
                   src_va, dst_va, src_vb, dst_vb, buf_a, buf_b, acc_sh,
                   gsem_a, gsem_b, ssem_a, ssem_b):
        c = lax.axis_index("c")
        s = lax.axis_index("s")
        r0 = s * ROWS_PER_TILE
        pltpu.sync_copy(zeros_ref.at[pl.ds(r0, ROWS_PER_TILE)],
                        acc_sh.at[pl.ds(r0, ROWS_PER_TILE)])
        plsc.subcore_barrier()
        wid = c * NS + s

        @pl.loop(0, chunks_per_worker // 2)
        def _(j):
            base = (wid * chunks_per_worker + 2 * j) * CHUNK
            pltpu.sync_copy(src_hbm.at[pl.ds(base, CHUNK)], src_va)
            pltpu.sync_copy(dst_hbm.at[pl.ds(base, CHUNK)], dst_va)
            pltpu.async_copy(y_hbm.at[src_va], buf_a, gsem_a).wait()
            ha = pltpu.async_copy(buf_a, acc_sh.at[dst_va], ssem_a, add=True)
            pltpu.sync_copy(src_hbm.at[pl.ds(base + CHUNK, CHUNK)], src_vb)
            pltpu.sync_copy(dst_hbm.at[pl.ds(base + CHUNK, CHUNK)], dst_vb)
            pltpu.async_copy(y_hbm.at[src_vb], buf_b, gsem_b).wait()
            hb = pltpu.async_copy(buf_b, acc_sh.at[dst_vb], ssem_b, add=True)
            ha.wait()
            hb.wait()

        if chunks_per_worker % 2:
            tail = (wid * chunks_per_worker + chunks_per_worker - 1) * CHUNK
            pltpu.sync_copy(src_hbm.at[pl.ds(tail, CHUNK)], src_va)
            pltpu.sync_copy(dst_hbm.at[pl.ds(tail, CHUNK)], dst_va)
            pltpu.async_copy(y_hbm.at[src_va], buf_a, gsem_a).wait()
            pltpu.sync_copy(buf_a, acc_sh.at[dst_va], add=True)

        plsc.subcore_barrier()
        pltpu.sync_copy(acc_sh.at[pl.ds(r0, ROWS_PER_TILE)],
                        out_hbm.at[pl.ds(c * NPAD + r0, ROWS_PER_TILE)])

    return agg_kernel(y, src_pad, dst_pad, zeros_hbm)


R = 1024
GRID = NPAD // R


def _dinv_of(deg_blk):
    return lax.rsqrt(deg_blk[0, :, 0:1] + deg_blk[1, :, 0:1] + 1.0)


def _tc_first(x_pad, W1, degs):
    def body(x_ref, w_ref, deg_ref, y_ref):
        dinv = _dinv_of(deg_ref[...])
        y = lax.dot_general(x_ref[...], w_ref[...], (((1,), (0,)), ((), ())),
                            precision=lax.Precision.HIGHEST,
                            preferred_element_type=jnp.float32)
        y_ref[...] = y * dinv

    return pl.pallas_call(
        body,
        grid=(GRID,),
        in_specs=[
            pl.BlockSpec((R, F), lambda i: (i, 0)),
            pl.BlockSpec((F, F), lambda i: (0, 0)),
            pl.BlockSpec((NC, R, DEG_W), lambda i: (0, i, 0)),
        ],
        out_specs=pl.BlockSpec((R, F), lambda i: (i, 0)),
        out_shape=jax.ShapeDtypeStruct((NPAD, F), jnp.float32),
    )(x_pad, W1, degs)


def _tc_middle(agg1, y1, degs, W2, b1):
    def body(a_ref, y1_ref, deg_ref, w_ref, b_ref, y2_ref):
        dinv = _dinv_of(deg_ref[...])
        a = a_ref[...]
        pre = dinv * (a[0] + a[1] + y1_ref[...]) + b_ref[...]
        h = jnp.maximum(pre, 0.0)
        y2 = lax.dot_general(h, w_ref[...], (((1,), (0,)), ((), ())),
                             precision=lax.Precision.HIGHEST,
                             preferred_element_type=jnp.float32)
        y2_ref[...] = y2 * dinv

    return pl.pallas_call(
        body,
        grid=(GRID,),
        in_specs=[
            pl.BlockSpec((NC, R, F), lambda i: (0, i, 0)),
            pl.BlockSpec((R, F), lambda i: (i, 0)),
            pl.BlockSpec((NC, R, DEG_W), lambda i: (0, i, 0)),
            pl.BlockSpec((F, F), lambda i: (0, 0)),
            pl.BlockSpec((1, F), lambda i: (0, 0)),
        ],
        out_specs=pl.BlockSpec((R, F), lambda i: (i, 0)),
        out_shape=jax.ShapeDtypeStruct((NPAD, F), jnp.float32),
    )(agg1, y1, degs, W2, b1)


def _tc_last(agg2, y2, degs, b2):
    def body(a_ref, y2_ref, deg_ref, b_ref, o_ref):
        dinv = _dinv_of(deg_ref[...])
        a = a_ref[...]
        pre = dinv * (a[0] + a[1] + y2_ref[...]) + b_ref[...]
        o_ref[...] = jax.nn.sigmoid(pre)

    return pl.pallas_call(
        body,
        grid=(GRID,),
        in_specs=[
            pl.BlockSpec((NC, R, F), lambda i: (0, i, 0)),
            pl.BlockSpec((R, F), lambda i: (i, 0)),
            pl.BlockSpec((NC, R, DEG_W), lambda i: (0, i, 0)),
            pl.BlockSpec((1, F), lambda i: (0, 0)),
        ],
        out_specs=pl.BlockSpec((R, F), lambda i: (i, 0)),
        out_shape=jax.ShapeDtypeStruct((NPAD, F), jnp.float32),
    )(agg2, y2, degs, b2)


def kernel(x, edge_index, W1, b1, W2, b2):
    src = edge_index[0].astype(jnp.int32)
    dst = edge_index[1].astype(jnp.int32)
    n_edges = src.shape[0]
    total_chunks = -(-n_edges // CHUNK)
    total_chunks = -(-total_chunks // NW) * NW
    e_pad = total_chunks * CHUNK - n_edges
    cpw = total_chunks // NW
    src_pad = jnp.concatenate([src, jnp.full((e_pad,), PAD_NODE, jnp.int32)])
    dst_pad = jnp.concatenate([dst, jnp.full((e_pad,), PAD_NODE, jnp.int32)])

    x_pad = jnp.pad(x, ((0, NPAD - x.shape[0]), (0, 0)))
    ones_hbm = jnp.ones((CHUNK, DEG_W), jnp.float32)
    zeros_deg = jnp.zeros((NPAD, DEG_W), jnp.float32)
    zeros_f = jnp.zeros((NPAD, F), jnp.float32)

    degs = _sc_degree(dst_pad, ones_hbm, zeros_deg, cpw).reshape(NC, NPAD, DEG_W)
    y1 = _tc_first(x_pad, W1, degs)
    agg1 = _sc_aggregate(y1, src_pad, dst_pad, zeros_f, cpw).reshape(NC, NPAD, F)
    y2 = _tc_middle(agg1, y1, degs, W2, b1.reshape(1, F))
    agg2 = _sc_aggregate(y2, src_pad, dst_pad, zeros_f, cpw).reshape(NC, NPAD, F)
    out = _tc_last(agg2, y2, degs, b2.reshape(1, F))
    return out[:N_NODES]


# speedup vs baseline: 1.4341x; 1.0729x over previous
import functools

import jax
import jax.numpy as jnp
from jax import lax
from jax.experimental import pallas as pl
from jax.experimental.pallas import tpu as pltpu
from jax.experimental.pallas import tpu_sc as plsc

N_NODES = 10000
F = 128
NPAD = 10240
PAD_NODE = N_NODES
DEG_W = 128
CHUNK = 128
NC = 2
NS = 16
NW = NC * NS
ROWS_PER_TILE = NPAD // NS

_mesh = plsc.VectorSubcoreMesh(core_axis_name="c", subcore_axis_name="s")


def _sc_degree(dst_pad, ones_hbm, zeros_hbm, chunks_per_worker):
    @functools.partial(
        pl.kernel,
        out_type=jax.ShapeDtypeStruct((NC * NPAD, DEG_W), jnp.float32),
        mesh=_mesh,
        scratch_types=[
            pltpu.VMEM((CHUNK,), jnp.int32),
            pltpu.VMEM((CHUNK,), jnp.int32),
            pltpu.VMEM((CHUNK, DEG_W), jnp.float32),
            pltpu.VMEM_SHARED((NPAD, DEG_W), jnp.float32),
            pltpu.SemaphoreType.DMA,
            pltpu.SemaphoreType.DMA,
        ],
    )
    def deg_kernel(dst_hbm, ones_ref, zeros_ref, out_hbm, dst_va, dst_vb,
                   ones_v, acc_sh, sema, semb):
        c = lax.axis_index("c")
        s = lax.axis_index("s")
        r0 = s * ROWS_PER_TILE
        pltpu.sync_copy(zeros_ref.at[pl.ds(r0, ROWS_PER_TILE)],
                        acc_sh.at[pl.ds(r0, ROWS_PER_TILE)])
        pltpu.sync_copy(ones_ref, ones_v)
        plsc.subcore_barrier()
        wid = c * NS + s

        @pl.loop(0, chunks_per_worker // 2)
        def _(j):
            base = (wid * chunks_per_worker + 2 * j) * CHUNK
            pltpu.sync_copy(dst_hbm.at[pl.ds(base, CHUNK)], dst_va)
            ha = pltpu.async_copy(ones_v, acc_sh.at[dst_va], sema, add=True)
            pltpu.sync_copy(dst_hbm.at[pl.ds(base + CHUNK, CHUNK)], dst_vb)
            hb = pltpu.async_copy(ones_v, acc_sh.at[dst_vb], semb, add=True)
            ha.wait()
            hb.wait()

        if chunks_per_worker % 2:
            tail = (wid * chunks_per_worker + chunks_per_worker - 1) * CHUNK
            pltpu.sync_copy(dst_hbm.at[pl.ds(tail, CHUNK)], dst_va)
            pltpu.sync_copy(ones_v, acc_sh.at[dst_va], add=True)

        plsc.subcore_barrier()
        pltpu.sync_copy(acc_sh.at[pl.ds(r0, ROWS_PER_TILE)],
                        out_hbm.at[pl.ds(c * NPAD + r0, ROWS_PER_TILE)])

    return deg_kernel(dst_pad, ones_hbm, zeros_hbm)


def _sc_aggregate(y, src_pad, dst_pad, zeros_hbm, chunks_per_worker):
    @functools.partial(
        pl.kernel,
        out_type=jax.ShapeDtypeStruct((NC * NPAD, F), jnp.float32),
        mesh=_mesh,
        scratch_types=[
            pltpu.VMEM((CHUNK,), jnp.int32),
            pltpu.VMEM((CHUNK,), jnp.int32),
            pltpu.VMEM((CHUNK,), jnp.int32),
            pltpu.VMEM((CHUNK,), jnp.int32),
            pltpu.VMEM((CHUNK, F), jnp.float32),
            pltpu.VMEM((CHUNK, F), jnp.float32),
            pltpu.VMEM_SHARED((NPAD, F), jnp.float32),
            pltpu.SemaphoreType.DMA,
            pltpu.SemaphoreType.DMA,
            pltpu.SemaphoreType.DMA,
            pltpu.SemaphoreType.DMA,
        ],
    )
    def agg_kernel(y_hbm, src_hbm, dst_hbm, zeros_ref, out_hbm,
                   src_va, dst_va, src_vb, dst_vb, buf_a, buf_b, acc_sh,
                   gsem_a, gsem_b, ssem_a, ssem_b):
        c = lax.axis_index("c")
        s = lax.axis_index("s")
        r0 = s * ROWS_PER_TILE
        pltpu.sync_copy(zeros_ref.at[pl.ds(r0, ROWS_PER_TILE)],
                        acc_sh.at[pl.ds(r0, ROWS_PER_TILE)])
        plsc.subcore_barrier()
        wid = c * NS + s

        @pl.loop(0, chunks_per_worker // 2)
        def _(j):
            base = (wid * chunks_per_worker + 2 * j) * CHUNK
            pltpu.sync_copy(src_hbm.at[pl.ds(base, CHUNK)], src_va)
            pltpu.sync_copy(dst_hbm.at[pl.ds(base, CHUNK)], dst_va)
            ga = pltpu.async_copy(y_hbm.at[src_va], buf_a, gsem_a)
            pltpu.sync_copy(src_hbm.at[pl.ds(base + CHUNK, CHUNK)], src_vb)
            pltpu.sync_copy(dst_hbm.at[pl.ds(base + CHUNK, CHUNK)], dst_vb)
            ga.wait()
            ha = pltpu.async_copy(buf_a, acc_sh.at[dst_va], ssem_a, add=True)
            pltpu.async_copy(y_hbm.at[src_vb], buf_b, gsem_b).wait()
            hb = pltpu.async_copy(buf_b, acc_sh.at[dst_vb], ssem_b, add=True)
            ha.wait()
            hb.wait()

        if chunks_per_worker % 2:
            tail = (wid * chunks_per_worker + chunks_per_worker - 1) * CHUNK
            pltpu.sync_copy(src_hbm.at[pl.ds(tail, CHUNK)], src_va)
            pltpu.sync_copy(dst_hbm.at[pl.ds(tail, CHUNK)], dst_va)
            pltpu.async_copy(y_hbm.at[src_va], buf_a, gsem_a).wait()
            pltpu.sync_copy(buf_a, acc_sh.at[dst_va], add=True)

        plsc.subcore_barrier()
        pltpu.sync_copy(acc_sh.at[pl.ds(r0, ROWS_PER_TILE)],
                        out_hbm.at[pl.ds(c * NPAD + r0, ROWS_PER_TILE)])

    return agg_kernel(y, src_pad, dst_pad, zeros_hbm)


R = 1024
GRID = NPAD // R


def _dinv_of(deg_blk):
    return lax.rsqrt(deg_blk[0, :, 0:1] + deg_blk[1, :, 0:1] + 1.0)


def _tc_first(x_pad, W1, degs):
    def body(x_ref, w_ref, deg_ref, y_ref):
        dinv = _dinv_of(deg_ref[...])
        y = lax.dot_general(x_ref[...], w_ref[...], (((1,), (0,)), ((), ())),
                            precision=lax.Precision.HIGHEST,
                            preferred_element_type=jnp.float32)
        y_ref[...] = y * dinv

    return pl.pallas_call(
        body,
        grid=(GRID,),
        in_specs=[
            pl.BlockSpec((R, F), lambda i: (i, 0)),
            pl.BlockSpec((F, F), lambda i: (0, 0)),
            pl.BlockSpec((NC, R, DEG_W), lambda i: (0, i, 0)),
        ],
        out_specs=pl.BlockSpec((R, F), lambda i: (i, 0)),
        out_shape=jax.ShapeDtypeStruct((NPAD, F), jnp.float32),
    )(x_pad, W1, degs)


def _tc_middle(agg1, y1, degs, W2, b1):
    def body(a_ref, y1_ref, deg_ref, w_ref, b_ref, y2_ref):
        dinv = _dinv_of(deg_ref[...])
        a = a_ref[...]
        pre = dinv * (a[0] + a[1] + y1_ref[...]) + b_ref[...]
        h = jnp.maximum(pre, 0.0)
        y2 = lax.dot_general(h, w_ref[...], (((1,), (0,)), ((), ())),
                             precision=lax.Precision.HIGHEST,
                             preferred_element_type=jnp.float32)
        y2_ref[...] = y2 * dinv

    return pl.pallas_call(
        body,
        grid=(GRID,),
        in_specs=[
            pl.BlockSpec((NC, R, F), lambda i: (0, i, 0)),
            pl.BlockSpec((R, F), lambda i: (i, 0)),
            pl.BlockSpec((NC, R, DEG_W), lambda i: (0, i, 0)),
            pl.BlockSpec((F, F), lambda i: (0, 0)),
            pl.BlockSpec((1, F), lambda i: (0, 0)),
        ],
        out_specs=pl.BlockSpec((R, F), lambda i: (i, 0)),
        out_shape=jax.ShapeDtypeStruct((NPAD, F), jnp.float32),
    )(agg1, y1, degs, W2, b1)


def _tc_last(agg2, y2, degs, b2):
    def body(a_ref, y2_ref, deg_ref, b_ref, o_ref):
        dinv = _dinv_of(deg_ref[...])
        a = a_ref[...]
        pre = dinv * (a[0] + a[1] + y2_ref[...]) + b_ref[...]
        o_ref[...] = jax.nn.sigmoid(pre)

    return pl.pallas_call(
        body,
        grid=(GRID,),
        in_specs=[
            pl.BlockSpec((NC, R, F), lambda i: (0, i, 0)),
            pl.BlockSpec((R, F), lambda i: (i, 0)),
            pl.BlockSpec((NC, R, DEG_W), lambda i: (0, i, 0)),
            pl.BlockSpec((1, F), lambda i: (0, 0)),
        ],
        out_specs=pl.BlockSpec((R, F), lambda i: (i, 0)),
        out_shape=jax.ShapeDtypeStruct((NPAD, F), jnp.float32),
    )(agg2, y2, degs, b2)


def kernel(x, edge_index, W1, b1, W2, b2):
    src = edge_index[0].astype(jnp.int32)
    dst = edge_index[1].astype(jnp.int32)
    n_edges = src.shape[0]
    total_chunks = -(-n_edges // CHUNK)
    total_chunks = -(-total_chunks // NW) * NW
    e_pad = total_chunks * CHUNK - n_edges
    cpw = total_chunks // NW
    src_pad = jnp.concatenate([src, jnp.full((e_pad,), PAD_NODE, jnp.int32)])
    dst_pad = jnp.concatenate([dst, jnp.full((e_pad,), PAD_NODE, jnp.int32)])

    x_pad = jnp.pad(x, ((0, NPAD - x.shape[0]), (0, 0)))
    ones_hbm = jnp.ones((CHUNK, DEG_W), jnp.float32)
    zeros_deg = jnp.zeros((NPAD, DEG_W), jnp.float32)
    zeros_f = jnp.zeros((NPAD, F), jnp.float32)

    degs = _sc_degree(dst_pad, ones_hbm, zeros_deg, cpw).reshape(NC, NPAD, DEG_W)
    y1 = _tc_first(x_pad, W1, degs)
    agg1 = _sc_aggregate(y1, src_pad, dst_pad, zeros_f, cpw).reshape(NC, NPAD, F)
    y2 = _tc_middle(agg1, y1, degs, W2, b1.reshape(1, F))
    agg2 = _sc_aggregate(y2, src_pad, dst_pad, zeros_f, cpw).reshape(NC, NPAD, F)
    out = _tc_last(agg2, y2, degs, b2.reshape(1, F))
    return out[:N_NODES]


# speedup vs baseline: 1.5407x; 1.0744x over previous
import functools

import jax
import jax.numpy as jnp
from jax import lax
from jax.experimental import pallas as pl
from jax.experimental.pallas import tpu as pltpu
from jax.experimental.pallas import tpu_sc as plsc

N_NODES = 10000
F = 128
NPAD = 10240
PAD_NODE = N_NODES
DEG_W = 128
CHUNK = 128
NC = 2
NS = 16
NW = NC * NS
ROWS_PER_TILE = NPAD // NS

_mesh = plsc.VectorSubcoreMesh(core_axis_name="c", subcore_axis_name="s")


def _sc_degree(dst_pad, ones_hbm, zeros_hbm, chunks_per_worker):
    @functools.partial(
        pl.kernel,
        out_type=jax.ShapeDtypeStruct((NC * NPAD, DEG_W), jnp.float32),
        mesh=_mesh,
        scratch_types=[
            pltpu.VMEM((CHUNK,), jnp.int32),
            pltpu.VMEM((CHUNK,), jnp.int32),
            pltpu.VMEM((CHUNK, DEG_W), jnp.float32),
            pltpu.VMEM_SHARED((NPAD, DEG_W), jnp.float32),
            pltpu.SemaphoreType.DMA,
            pltpu.SemaphoreType.DMA,
        ],
    )
    def deg_kernel(dst_hbm, ones_ref, zeros_ref, out_hbm, dst_va, dst_vb,
                   ones_v, acc_sh, sema, semb):
        c = lax.axis_index("c")
        s = lax.axis_index("s")
        r0 = s * ROWS_PER_TILE
        pltpu.sync_copy(zeros_ref.at[pl.ds(r0, ROWS_PER_TILE)],
                        acc_sh.at[pl.ds(r0, ROWS_PER_TILE)])
        pltpu.sync_copy(ones_ref, ones_v)
        plsc.subcore_barrier()
        wid = c * NS + s

        @pl.loop(0, chunks_per_worker // 2)
        def _(j):
            base = (wid * chunks_per_worker + 2 * j) * CHUNK
            pltpu.sync_copy(dst_hbm.at[pl.ds(base, CHUNK)], dst_va)
            ha = pltpu.async_copy(ones_v, acc_sh.at[dst_va], sema, add=True)
            pltpu.sync_copy(dst_hbm.at[pl.ds(base + CHUNK, CHUNK)], dst_vb)
            hb = pltpu.async_copy(ones_v, acc_sh.at[dst_vb], semb, add=True)
            ha.wait()
            hb.wait()

        if chunks_per_worker % 2:
            tail = (wid * chunks_per_worker + chunks_per_worker - 1) * CHUNK
            pltpu.sync_copy(dst_hbm.at[pl.ds(tail, CHUNK)], dst_va)
            pltpu.sync_copy(ones_v, acc_sh.at[dst_va], add=True)

        plsc.subcore_barrier()
        pltpu.sync_copy(acc_sh.at[pl.ds(r0, ROWS_PER_TILE)],
                        out_hbm.at[pl.ds(c * NPAD + r0, ROWS_PER_TILE)])

    return deg_kernel(dst_pad, ones_hbm, zeros_hbm)


def _sc_aggregate(y, src_pad, dst_pad, zeros_hbm, chunks_per_worker):
    @functools.partial(
        pl.kernel,
        out_type=jax.ShapeDtypeStruct((NC * NPAD, F), jnp.float32),
        mesh=_mesh,
        scratch_types=[
            pltpu.VMEM((CHUNK,), jnp.int32),
            pltpu.VMEM((CHUNK,), jnp.int32),
            pltpu.VMEM((CHUNK,), jnp.int32),
            pltpu.VMEM((CHUNK,), jnp.int32),
            pltpu.VMEM((CHUNK, F), jnp.float32),
            pltpu.VMEM((CHUNK, F), jnp.float32),
            pltpu.VMEM_SHARED((NPAD, F), jnp.float32),
            pltpu.SemaphoreType.DMA,
            pltpu.SemaphoreType.DMA,
            pltpu.SemaphoreType.DMA,
            pltpu.SemaphoreType.DMA,
        ],
    )
    def agg_kernel(y_hbm, src_hbm, dst_hbm, zeros_ref, out_hbm,
                   src_va, dst_va, src_vb, dst_vb, buf_a, buf_b, acc_sh,
                   gsem_a, gsem_b, ssem_a, ssem_b):
        c = lax.axis_index("c")
        s = lax.axis_index("s")
        r0 = s * ROWS_PER_TILE
        pltpu.sync_copy(zeros_ref.at[pl.ds(r0, ROWS_PER_TILE)],
                        acc_sh.at[pl.ds(r0, ROWS_PER_TILE)])
        plsc.subcore_barrier()
        wid = c * NS + s

        npairs = chunks_per_worker // 2
        base0 = wid * chunks_per_worker * CHUNK
        pltpu.sync_copy(src_hbm.at[pl.ds(base0, CHUNK)], src_va)
        pltpu.sync_copy(dst_hbm.at[pl.ds(base0, CHUNK)], dst_va)

        @pl.loop(0, npairs)
        def _(j):
            base = (wid * chunks_per_worker + 2 * j) * CHUNK
            ga = pltpu.async_copy(y_hbm.at[src_va], buf_a, gsem_a)
            pltpu.sync_copy(src_hbm.at[pl.ds(base + CHUNK, CHUNK)], src_vb)
            pltpu.sync_copy(dst_hbm.at[pl.ds(base + CHUNK, CHUNK)], dst_vb)
            ga.wait()
            ha = pltpu.async_copy(buf_a, acc_sh.at[dst_va], ssem_a, add=True)
            gb = pltpu.async_copy(y_hbm.at[src_vb], buf_b, gsem_b)

            @pl.when(j + 1 < npairs)
            def _():
                pltpu.sync_copy(src_hbm.at[pl.ds(base + 2 * CHUNK, CHUNK)],
                                src_va)

            gb.wait()
            ha.wait()
            hb = pltpu.async_copy(buf_b, acc_sh.at[dst_vb], ssem_b, add=True)

            @pl.when(j + 1 < npairs)
            def _():
                pltpu.sync_copy(dst_hbm.at[pl.ds(base + 2 * CHUNK, CHUNK)],
                                dst_va)

            hb.wait()

        if chunks_per_worker % 2:
            tail = (wid * chunks_per_worker + chunks_per_worker - 1) * CHUNK
            pltpu.sync_copy(src_hbm.at[pl.ds(tail, CHUNK)], src_va)
            pltpu.sync_copy(dst_hbm.at[pl.ds(tail, CHUNK)], dst_va)
            pltpu.async_copy(y_hbm.at[src_va], buf_a, gsem_a).wait()
            pltpu.sync_copy(buf_a, acc_sh.at[dst_va], add=True)

        plsc.subcore_barrier()
        pltpu.sync_copy(acc_sh.at[pl.ds(r0, ROWS_PER_TILE)],
                        out_hbm.at[pl.ds(c * NPAD + r0, ROWS_PER_TILE)])

    return agg_kernel(y, src_pad, dst_pad, zeros_hbm)


R = 1024
GRID = NPAD // R


def _dinv_of(deg_blk):
    return lax.rsqrt(deg_blk[0, :, 0:1] + deg_blk[1, :, 0:1] + 1.0)


def _tc_first(x_pad, W1, degs):
    def body(x_ref, w_ref, deg_ref, y_ref):
        dinv = _dinv_of(deg_ref[...])
        y = lax.dot_general(x_ref[...], w_ref[...], (((1,), (0,)), ((), ())),
                            precision=lax.Precision.HIGHEST,
                            preferred_element_type=jnp.float32)
        y_ref[...] = y * dinv

    return pl.pallas_call(
        body,
        grid=(GRID,),
        in_specs=[
            pl.BlockSpec((R, F), lambda i: (i, 0)),
            pl.BlockSpec((F, F), lambda i: (0, 0)),
            pl.BlockSpec((NC, R, DEG_W), lambda i: (0, i, 0)),
        ],
        out_specs=pl.BlockSpec((R, F), lambda i: (i, 0)),
        out_shape=jax.ShapeDtypeStruct((NPAD, F), jnp.float32),
    )(x_pad, W1, degs)


def _tc_middle(agg1, y1, degs, W2, b1):
    def body(a_ref, y1_ref, deg_ref, w_ref, b_ref, y2_ref):
        dinv = _dinv_of(deg_ref[...])
        a = a_ref[...]
        pre = dinv * (a[0] + a[1] + y1_ref[...]) + b_ref[...]
        h = jnp.maximum(pre, 0.0)
        y2 = lax.dot_general(h, w_ref[...], (((1,), (0,)), ((), ())),
                             precision=lax.Precision.HIGHEST,
                             preferred_element_type=jnp.float32)
        y2_ref[...] = y2 * dinv

    return pl.pallas_call(
        body,
        grid=(GRID,),
        in_specs=[
            pl.BlockSpec((NC, R, F), lambda i: (0, i, 0)),
            pl.BlockSpec((R, F), lambda i: (i, 0)),
            pl.BlockSpec((NC, R, DEG_W), lambda i: (0, i, 0)),
            pl.BlockSpec((F, F), lambda i: (0, 0)),
            pl.BlockSpec((1, F), lambda i: (0, 0)),
        ],
        out_specs=pl.BlockSpec((R, F), lambda i: (i, 0)),
        out_shape=jax.ShapeDtypeStruct((NPAD, F), jnp.float32),
    )(agg1, y1, degs, W2, b1)


def _tc_last(agg2, y2, degs, b2):
    def body(a_ref, y2_ref, deg_ref, b_ref, o_ref):
        dinv = _dinv_of(deg_ref[...])
        a = a_ref[...]
        pre = dinv * (a[0] + a[1] + y2_ref[...]) + b_ref[...]
        o_ref[...] = jax.nn.sigmoid(pre)

    return pl.pallas_call(
        body,
        grid=(GRID,),
        in_specs=[
            pl.BlockSpec((NC, R, F), lambda i: (0, i, 0)),
            pl.BlockSpec((R, F), lambda i: (i, 0)),
            pl.BlockSpec((NC, R, DEG_W), lambda i: (0, i, 0)),
            pl.BlockSpec((1, F), lambda i: (0, 0)),
        ],
        out_specs=pl.BlockSpec((R, F), lambda i: (i, 0)),
        out_shape=jax.ShapeDtypeStruct((NPAD, F), jnp.float32),
    )(agg2, y2, degs, b2)


def kernel(x, edge_index, W1, b1, W2, b2):
    src = edge_index[0].astype(jnp.int32)
    dst = edge_index[1].astype(jnp.int32)
    n_edges = src.shape[0]
    total_chunks = -(-n_edges // CHUNK)
    total_chunks = -(-total_chunks // NW) * NW
    e_pad = total_chunks * CHUNK - n_edges
    cpw = total_chunks // NW
    src_pad = jnp.concatenate([src, jnp.full((e_pad,), PAD_NODE, jnp.int32)])
    dst_pad = jnp.concatenate([dst, jnp.full((e_pad,), PAD_NODE, jnp.int32)])

    x_pad = jnp.pad(x, ((0, NPAD - x.shape[0]), (0, 0)))
    ones_hbm = jnp.ones((CHUNK, DEG_W), jnp.float32)
    zeros_deg = jnp.zeros((NPAD, DEG_W), jnp.float32)
    zeros_f = jnp.zeros((NPAD, F), jnp.float32)

    degs = _sc_degree(dst_pad, ones_hbm, zeros_deg, cpw).reshape(NC, NPAD, DEG_W)
    y1 = _tc_first(x_pad, W1, degs)
    agg1 = _sc_aggregate(y1, src_pad, dst_pad, zeros_f, cpw).reshape(NC, NPAD, F)
    y2 = _tc_middle(agg1, y1, degs, W2, b1.reshape(1, F))
    agg2 = _sc_aggregate(y2, src_pad, dst_pad, zeros_f, cpw).reshape(NC, NPAD, F)
    out = _tc_last(agg2, y2, degs, b2.reshape(1, F))
    return out[:N_NODES]


# speedup vs baseline: 1.5622x; 1.0139x over previous
import functools

import jax
import jax.numpy as jnp
from jax import lax
from jax.experimental import pallas as pl
from jax.experimental.pallas import tpu as pltpu
from jax.experimental.pallas import tpu_sc as plsc

N_NODES = 10000
F = 128
NPAD = 10240
PAD_NODE = N_NODES
DEG_W = 128
CHUNK = 128
NC = 2
NS = 16
NW = NC * NS
ROWS_PER_TILE = NPAD // NS

_mesh = plsc.VectorSubcoreMesh(core_axis_name="c", subcore_axis_name="s")


def _sc_degree(dst_pad, ones_hbm, zeros_hbm, chunks_per_worker):
    @functools.partial(
        pl.kernel,
        out_type=jax.ShapeDtypeStruct((NC * NPAD, DEG_W), jnp.float32),
        mesh=_mesh,
        scratch_types=[
            pltpu.VMEM((CHUNK,), jnp.int32),
            pltpu.VMEM((CHUNK,), jnp.int32),
            pltpu.VMEM((CHUNK, DEG_W), jnp.float32),
            pltpu.VMEM_SHARED((NPAD, DEG_W), jnp.float32),
            pltpu.SemaphoreType.DMA,
            pltpu.SemaphoreType.DMA,
        ],
    )
    def deg_kernel(dst_hbm, ones_ref, zeros_ref, out_hbm, dst_va, dst_vb,
                   ones_v, acc_sh, sema, semb):
        c = lax.axis_index("c")
        s = lax.axis_index("s")
        r0 = s * ROWS_PER_TILE
        pltpu.sync_copy(zeros_ref.at[pl.ds(r0, ROWS_PER_TILE)],
                        acc_sh.at[pl.ds(r0, ROWS_PER_TILE)])
        pltpu.sync_copy(ones_ref, ones_v)
        plsc.subcore_barrier()
        wid = c * NS + s

        npairs = chunks_per_worker // 2
        base0 = wid * chunks_per_worker * CHUNK
        pltpu.sync_copy(dst_hbm.at[pl.ds(base0, CHUNK)], dst_va)

        @pl.loop(0, npairs)
        def _(j):
            base = (wid * chunks_per_worker + 2 * j) * CHUNK
            ha = pltpu.async_copy(ones_v, acc_sh.at[dst_va], sema, add=True)
            pltpu.sync_copy(dst_hbm.at[pl.ds(base + CHUNK, CHUNK)], dst_vb)
            hb = pltpu.async_copy(ones_v, acc_sh.at[dst_vb], semb, add=True)
            ha.wait()

            @pl.when(j + 1 < npairs)
            def _():
                pltpu.sync_copy(dst_hbm.at[pl.ds(base + 2 * CHUNK, CHUNK)],
                                dst_va)

            hb.wait()

        if chunks_per_worker % 2:
            tail = (wid * chunks_per_worker + chunks_per_worker - 1) * CHUNK
            pltpu.sync_copy(dst_hbm.at[pl.ds(tail, CHUNK)], dst_va)
            pltpu.sync_copy(ones_v, acc_sh.at[dst_va], add=True)

        plsc.subcore_barrier()
        pltpu.sync_copy(acc_sh.at[pl.ds(r0, ROWS_PER_TILE)],
                        out_hbm.at[pl.ds(c * NPAD + r0, ROWS_PER_TILE)])

    return deg_kernel(dst_pad, ones_hbm, zeros_hbm)


def _sc_aggregate(y, src_pad, dst_pad, zeros_hbm, chunks_per_worker):
    @functools.partial(
        pl.kernel,
        out_type=jax.ShapeDtypeStruct((NC * NPAD, F), jnp.float32),
        mesh=_mesh,
        scratch_types=[
            pltpu.VMEM((CHUNK,), jnp.int32),
            pltpu.VMEM((CHUNK,), jnp.int32),
            pltpu.VMEM((CHUNK,), jnp.int32),
            pltpu.VMEM((CHUNK,), jnp.int32),
            pltpu.VMEM((CHUNK, F), jnp.float32),
            pltpu.VMEM((CHUNK, F), jnp.float32),
            pltpu.VMEM_SHARED((NPAD, F), jnp.float32),
            pltpu.SemaphoreType.DMA,
            pltpu.SemaphoreType.DMA,
            pltpu.SemaphoreType.DMA,
            pltpu.SemaphoreType.DMA,
        ],
    )
    def agg_kernel(y_hbm, src_hbm, dst_hbm, zeros_ref, out_hbm,
                   src_va, dst_va, src_vb, dst_vb, buf_a, buf_b, acc_sh,
                   gsem_a, gsem_b, ssem_a, ssem_b):
        c = lax.axis_index("c")
        s = lax.axis_index("s")
        r0 = s * ROWS_PER_TILE
        pltpu.sync_copy(zeros_ref.at[pl.ds(r0, ROWS_PER_TILE)],
                        acc_sh.at[pl.ds(r0, ROWS_PER_TILE)])
        plsc.subcore_barrier()
        wid = c * NS + s

        npairs = chunks_per_worker // 2
        base0 = wid * chunks_per_worker * CHUNK
        pltpu.sync_copy(src_hbm.at[pl.ds(base0, CHUNK)], src_va)
        pltpu.sync_copy(dst_hbm.at[pl.ds(base0, CHUNK)], dst_va)

        @pl.loop(0, npairs)
        def _(j):
            base = (wid * chunks_per_worker + 2 * j) * CHUNK
            ga = pltpu.async_copy(y_hbm.at[src_va], buf_a, gsem_a)
            pltpu.sync_copy(src_hbm.at[pl.ds(base + CHUNK, CHUNK)], src_vb)
            pltpu.sync_copy(dst_hbm.at[pl.ds(base + CHUNK, CHUNK)], dst_vb)
            ga.wait()
            ha = pltpu.async_copy(buf_a, acc_sh.at[dst_va], ssem_a, add=True)
            gb = pltpu.async_copy(y_hbm.at[src_vb], buf_b, gsem_b)

            @pl.when(j + 1 < npairs)
            def _():
                pltpu.sync_copy(src_hbm.at[pl.ds(base + 2 * CHUNK, CHUNK)],
                                src_va)

            gb.wait()
            ha.wait()
            hb = pltpu.async_copy(buf_b, acc_sh.at[dst_vb], ssem_b, add=True)

            @pl.when(j + 1 < npairs)
            def _():
                pltpu.sync_copy(dst_hbm.at[pl.ds(base + 2 * CHUNK, CHUNK)],
                                dst_va)

            hb.wait()

        if chunks_per_worker % 2:
            tail = (wid * chunks_per_worker + chunks_per_worker - 1) * CHUNK
            pltpu.sync_copy(src_hbm.at[pl.ds(tail, CHUNK)], src_va)
            pltpu.sync_copy(dst_hbm.at[pl.ds(tail, CHUNK)], dst_va)
            pltpu.async_copy(y_hbm.at[src_va], buf_a, gsem_a).wait()
            pltpu.sync_copy(buf_a, acc_sh.at[dst_va], add=True)

        plsc.subcore_barrier()
        pltpu.sync_copy(acc_sh.at[pl.ds(r0, ROWS_PER_TILE)],
                        out_hbm.at[pl.ds(c * NPAD + r0, ROWS_PER_TILE)])

    return agg_kernel(y, src_pad, dst_pad, zeros_hbm)


R = 1024
GRID = NPAD // R


def _dinv_of(deg_blk):
    return lax.rsqrt(deg_blk[0, :, 0:1] + deg_blk[1, :, 0:1] + 1.0)


def _tc_first(x_pad, W1, degs):
    def body(x_ref, w_ref, deg_ref, y_ref):
        dinv = _dinv_of(deg_ref[...])
        y = lax.dot_general(x_ref[...], w_ref[...], (((1,), (0,)), ((), ())),
                            precision=lax.Precision.HIGHEST,
                            preferred_element_type=jnp.float32)
        y_ref[...] = y * dinv

    return pl.pallas_call(
        body,
        grid=(GRID,),
        in_specs=[
            pl.BlockSpec((R, F), lambda i: (i, 0)),
            pl.BlockSpec((F, F), lambda i: (0, 0)),
            pl.BlockSpec((NC, R, DEG_W), lambda i: (0, i, 0)),
        ],
        out_specs=pl.BlockSpec((R, F), lambda i: (i, 0)),
        out_shape=jax.ShapeDtypeStruct((NPAD, F), jnp.float32),
    )(x_pad, W1, degs)


def _tc_middle(agg1, y1, degs, W2, b1):
    def body(a_ref, y1_ref, deg_ref, w_ref, b_ref, y2_ref):
        dinv = _dinv_of(deg_ref[...])
        a = a_ref[...]
        pre = dinv * (a[0] + a[1] + y1_ref[...]) + b_ref[...]
        h = jnp.maximum(pre, 0.0)
        y2 = lax.dot_general(h, w_ref[...], (((1,), (0,)), ((), ())),
                             precision=lax.Precision.HIGHEST,
                             preferred_element_type=jnp.float32)
        y2_ref[...] = y2 * dinv

    return pl.pallas_call(
        body,
        grid=(GRID,),
        in_specs=[
            pl.BlockSpec((NC, R, F), lambda i: (0, i, 0)),
            pl.BlockSpec((R, F), lambda i: (i, 0)),
            pl.BlockSpec((NC, R, DEG_W), lambda i: (0, i, 0)),
            pl.BlockSpec((F, F), lambda i: (0, 0)),
            pl.BlockSpec((1, F), lambda i: (0, 0)),
        ],
        out_specs=pl.BlockSpec((R, F), lambda i: (i, 0)),
        out_shape=jax.ShapeDtypeStruct((NPAD, F), jnp.float32),
    )(agg1, y1, degs, W2, b1)


def _tc_last(agg2, y2, degs, b2):
    def body(a_ref, y2_ref, deg_ref, b_ref, o_ref):
        dinv = _dinv_of(deg_ref[...])
        a = a_ref[...]
        pre = dinv * (a[0] + a[1] + y2_ref[...]) + b_ref[...]
        o_ref[...] = jax.nn.sigmoid(pre)

    return pl.pallas_call(
        body,
        grid=(GRID,),
        in_specs=[
            pl.BlockSpec((NC, R, F), lambda i: (0, i, 0)),
            pl.BlockSpec((R, F), lambda i: (i, 0)),
            pl.BlockSpec((NC, R, DEG_W), lambda i: (0, i, 0)),
            pl.BlockSpec((1, F), lambda i: (0, 0)),
        ],
        out_specs=pl.BlockSpec((R, F), lambda i: (i, 0)),
        out_shape=jax.ShapeDtypeStruct((NPAD, F), jnp.float32),
    )(agg2, y2, degs, b2)


def kernel(x, edge_index, W1, b1, W2, b2):
    src = edge_index[0].astype(jnp.int32)
    dst = edge_index[1].astype(jnp.int32)
    n_edges = src.shape[0]
    total_chunks = -(-n_edges // CHUNK)
    total_chunks = -(-total_chunks // NW) * NW
    e_pad = total_chunks * CHUNK - n_edges
    cpw = total_chunks // NW
    src_pad = jnp.concatenate([src, jnp.full((e_pad,), PAD_NODE, jnp.int32)])
    dst_pad = jnp.concatenate([dst, jnp.full((e_pad,), PAD_NODE, jnp.int32)])

    x_pad = jnp.pad(x, ((0, NPAD - x.shape[0]), (0, 0)))
    ones_hbm = jnp.ones((CHUNK, DEG_W), jnp.float32)
    zeros_deg = jnp.zeros((NPAD, DEG_W), jnp.float32)
    zeros_f = jnp.zeros((NPAD, F), jnp.float32)

    degs = _sc_degree(dst_pad, ones_hbm, zeros_deg, cpw).reshape(NC, NPAD, DEG_W)
    y1 = _tc_first(x_pad, W1, degs)
    agg1 = _sc_aggregate(y1, src_pad, dst_pad, zeros_f, cpw).reshape(NC, NPAD, F)
    y2 = _tc_middle(agg1, y1, degs, W2, b1.reshape(1, F))
    agg2 = _sc_aggregate(y2, src_pad, dst_pad, zeros_f, cpw).reshape(NC, NPAD, F)
    out = _tc_last(agg2, y2, degs, b2.reshape(1, F))
    return out[:N_NODES]


# speedup vs baseline: 1.6076x; 1.0291x over previous
import functools

import jax
import jax.numpy as jnp
from jax import lax
from jax.experimental import pallas as pl
from jax.experimental.pallas import tpu as pltpu
from jax.experimental.pallas import tpu_sc as plsc

N_NODES = 10000
F = 128
NPAD = 10240
PAD_NODE = N_NODES
DEG_W = 128
CHUNK = 128
NC = 2
NS = 16
NW = NC * NS
ROWS_PER_TILE = NPAD // NS

_mesh = plsc.VectorSubcoreMesh(core_axis_name="c", subcore_axis_name="s")


def _sc_degree(dst_pad, ones_hbm, zeros_hbm, chunks_per_worker):
    @functools.partial(
        pl.kernel,
        out_type=jax.ShapeDtypeStruct((NC * NPAD, DEG_W), jnp.float32),
        mesh=_mesh,
        scratch_types=[
            pltpu.VMEM((CHUNK,), jnp.int32),
            pltpu.VMEM((CHUNK,), jnp.int32),
            pltpu.VMEM((CHUNK, DEG_W), jnp.float32),
            pltpu.VMEM_SHARED((NPAD, DEG_W), jnp.float32),
            pltpu.SemaphoreType.DMA,
            pltpu.SemaphoreType.DMA,
        ],
    )
    def deg_kernel(dst_hbm, ones_ref, zeros_ref, out_hbm, dst_va, dst_vb,
                   ones_v, acc_sh, sema, semb):
        c = lax.axis_index("c")
        s = lax.axis_index("s")
        r0 = s * ROWS_PER_TILE
        pltpu.sync_copy(zeros_ref.at[pl.ds(r0, ROWS_PER_TILE)],
                        acc_sh.at[pl.ds(r0, ROWS_PER_TILE)])
        pltpu.sync_copy(ones_ref, ones_v)
        plsc.subcore_barrier()
        wid = c * NS + s

        npairs = chunks_per_worker // 2
        base0 = wid * chunks_per_worker * CHUNK
        pltpu.sync_copy(dst_hbm.at[pl.ds(base0, CHUNK)], dst_va)

        @pl.loop(0, npairs)
        def _(j):
            base = (wid * chunks_per_worker + 2 * j) * CHUNK
            ha = pltpu.async_copy(ones_v, acc_sh.at[dst_va], sema, add=True)
            pltpu.sync_copy(dst_hbm.at[pl.ds(base + CHUNK, CHUNK)], dst_vb)
            hb = pltpu.async_copy(ones_v, acc_sh.at[dst_vb], semb, add=True)
            ha.wait()

            @pl.when(j + 1 < npairs)
            def _():
                pltpu.sync_copy(dst_hbm.at[pl.ds(base + 2 * CHUNK, CHUNK)],
                                dst_va)

            hb.wait()

        if chunks_per_worker % 2:
            tail = (wid * chunks_per_worker + chunks_per_worker - 1) * CHUNK
            pltpu.sync_copy(dst_hbm.at[pl.ds(tail, CHUNK)], dst_va)
            pltpu.sync_copy(ones_v, acc_sh.at[dst_va], add=True)

        plsc.subcore_barrier()
        pltpu.sync_copy(acc_sh.at[pl.ds(r0, ROWS_PER_TILE)],
                        out_hbm.at[pl.ds(c * NPAD + r0, ROWS_PER_TILE)])

    return deg_kernel(dst_pad, ones_hbm, zeros_hbm)


def _sc_aggregate(y, src_pad, dst_pad, zeros_hbm, chunks_per_worker):
    @functools.partial(
        pl.kernel,
        out_type=jax.ShapeDtypeStruct((NC * NPAD, F), jnp.float32),
        mesh=_mesh,
        scratch_types=[
            pltpu.VMEM((CHUNK,), jnp.int32),
            pltpu.VMEM((CHUNK,), jnp.int32),
            pltpu.VMEM((CHUNK,), jnp.int32),
            pltpu.VMEM((CHUNK,), jnp.int32),
            pltpu.VMEM((CHUNK, F), jnp.float32),
            pltpu.VMEM((CHUNK, F), jnp.float32),
            pltpu.VMEM_SHARED((NPAD, F), jnp.float32),
            pltpu.SemaphoreType.DMA,
            pltpu.SemaphoreType.DMA,
            pltpu.SemaphoreType.DMA,
            pltpu.SemaphoreType.DMA,
        ],
    )
    def agg_kernel(y_hbm, src_hbm, dst_hbm, zeros_ref, out_hbm,
                   src_va, dst_va, src_vb, dst_vb, buf_a, buf_b, acc_sh,
                   gsem_a, gsem_b, ssem_a, ssem_b):
        c = lax.axis_index("c")
        s = lax.axis_index("s")
        r0 = s * ROWS_PER_TILE
        pltpu.sync_copy(zeros_ref.at[pl.ds(r0, ROWS_PER_TILE)],
                        acc_sh.at[pl.ds(r0, ROWS_PER_TILE)])
        plsc.subcore_barrier()
        wid = c * NS + s

        npairs = chunks_per_worker // 2
        base0 = wid * chunks_per_worker * CHUNK
        pltpu.sync_copy(src_hbm.at[pl.ds(base0, CHUNK)], src_va)
        pltpu.sync_copy(dst_hbm.at[pl.ds(base0, CHUNK)], dst_va)

        @pl.loop(0, npairs)
        def _(j):
            base = (wid * chunks_per_worker + 2 * j) * CHUNK
            ga = pltpu.async_copy(y_hbm.at[src_va], buf_a, gsem_a)
            pltpu.sync_copy(src_hbm.at[pl.ds(base + CHUNK, CHUNK)], src_vb)
            pltpu.sync_copy(dst_hbm.at[pl.ds(base + CHUNK, CHUNK)], dst_vb)
            gb = pltpu.async_copy(y_hbm.at[src_vb], buf_b, gsem_b)
            ga.wait()
            ha = pltpu.async_copy(buf_a, acc_sh.at[dst_va], ssem_a, add=True)

            @pl.when(j + 1 < npairs)
            def _():
                pltpu.sync_copy(src_hbm.at[pl.ds(base + 2 * CHUNK, CHUNK)],
                                src_va)

            gb.wait()
            ha.wait()
            hb = pltpu.async_copy(buf_b, acc_sh.at[dst_vb], ssem_b, add=True)

            @pl.when(j + 1 < npairs)
            def _():
                pltpu.sync_copy(dst_hbm.at[pl.ds(base + 2 * CHUNK, CHUNK)],
                                dst_va)

            hb.wait()

        if chunks_per_worker % 2:
            tail = (wid * chunks_per_worker + chunks_per_worker - 1) * CHUNK
            pltpu.sync_copy(src_hbm.at[pl.ds(tail, CHUNK)], src_va)
            pltpu.sync_copy(dst_hbm.at[pl.ds(tail, CHUNK)], dst_va)
            pltpu.async_copy(y_hbm.at[src_va], buf_a, gsem_a).wait()
            pltpu.sync_copy(buf_a, acc_sh.at[dst_va], add=True)

        plsc.subcore_barrier()
        pltpu.sync_copy(acc_sh.at[pl.ds(r0, ROWS_PER_TILE)],
                        out_hbm.at[pl.ds(c * NPAD + r0, ROWS_PER_TILE)])

    return agg_kernel(y, src_pad, dst_pad, zeros_hbm)


R = 1024
GRID = NPAD // R


def _dinv_of(deg_blk):
    return lax.rsqrt(deg_blk[0, :, 0:1] + deg_blk[1, :, 0:1] + 1.0)


def _tc_first(x_pad, W1, degs):
    def body(x_ref, w_ref, deg_ref, y_ref):
        dinv = _dinv_of(deg_ref[...])
        y = lax.dot_general(x_ref[...], w_ref[...], (((1,), (0,)), ((), ())),
                            precision=lax.Precision.HIGHEST,
                            preferred_element_type=jnp.float32)
        y_ref[...] = y * dinv

    return pl.pallas_call(
        body,
        grid=(GRID,),
        in_specs=[
            pl.BlockSpec((R, F), lambda i: (i, 0)),
            pl.BlockSpec((F, F), lambda i: (0, 0)),
            pl.BlockSpec((NC, R, DEG_W), lambda i: (0, i, 0)),
        ],
        out_specs=pl.BlockSpec((R, F), lambda i: (i, 0)),
        out_shape=jax.ShapeDtypeStruct((NPAD, F), jnp.float32),
    )(x_pad, W1, degs)


def _tc_middle(agg1, y1, degs, W2, b1):
    def body(a_ref, y1_ref, deg_ref, w_ref, b_ref, y2_ref):
        dinv = _dinv_of(deg_ref[...])
        a = a_ref[...]
        pre = dinv * (a[0] + a[1] + y1_ref[...]) + b_ref[...]
        h = jnp.maximum(pre, 0.0)
        y2 = lax.dot_general(h, w_ref[...], (((1,), (0,)), ((), ())),
                             precision=lax.Precision.HIGHEST,
                             preferred_element_type=jnp.float32)
        y2_ref[...] = y2 * dinv

    return pl.pallas_call(
        body,
        grid=(GRID,),
        in_specs=[
            pl.BlockSpec((NC, R, F), lambda i: (0, i, 0)),
            pl.BlockSpec((R, F), lambda i: (i, 0)),
            pl.BlockSpec((NC, R, DEG_W), lambda i: (0, i, 0)),
            pl.BlockSpec((F, F), lambda i: (0, 0)),
            pl.BlockSpec((1, F), lambda i: (0, 0)),
        ],
        out_specs=pl.BlockSpec((R, F), lambda i: (i, 0)),
        out_shape=jax.ShapeDtypeStruct((NPAD, F), jnp.float32),
    )(agg1, y1, degs, W2, b1)


def _tc_last(agg2, y2, degs, b2):
    def body(a_ref, y2_ref, deg_ref, b_ref, o_ref):
        dinv = _dinv_of(deg_ref[...])
        a = a_ref[...]
        pre = dinv * (a[0] + a[1] + y2_ref[...]) + b_ref[...]
        o_ref[...] = jax.nn.sigmoid(pre)

    return pl.pallas_call(
        body,
        grid=(GRID,),
        in_specs=[
            pl.BlockSpec((NC, R, F), lambda i: (0, i, 0)),
            pl.BlockSpec((R, F), lambda i: (i, 0)),
            pl.BlockSpec((NC, R, DEG_W), lambda i: (0, i, 0)),
            pl.BlockSpec((1, F), lambda i: (0, 0)),
        ],
        out_specs=pl.BlockSpec((R, F), lambda i: (i, 0)),
        out_shape=jax.ShapeDtypeStruct((NPAD, F), jnp.float32),
    )(agg2, y2, degs, b2)


def kernel(x, edge_index, W1, b1, W2, b2):
    src = edge_index[0].astype(jnp.int32)
    dst = edge_index[1].astype(jnp.int32)
    n_edges = src.shape[0]
    total_chunks = -(-n_edges // CHUNK)
    total_chunks = -(-total_chunks // NW) * NW
    e_pad = total_chunks * CHUNK - n_edges
    cpw = total_chunks // NW
    src_pad = jnp.concatenate([src, jnp.full((e_pad,), PAD_NODE, jnp.int32)])
    dst_pad = jnp.concatenate([dst, jnp.full((e_pad,), PAD_NODE, jnp.int32)])

    x_pad = jnp.pad(x, ((0, NPAD - x.shape[0]), (0, 0)))
    ones_hbm = jnp.ones((CHUNK, DEG_W), jnp.float32)
    zeros_deg = jnp.zeros((NPAD, DEG_W), jnp.float32)
    zeros_f = jnp.zeros((NPAD, F), jnp.float32)

    degs = _sc_degree(dst_pad, ones_hbm, zeros_deg, cpw).reshape(NC, NPAD, DEG_W)
    y1 = _tc_first(x_pad, W1, degs)
    agg1 = _sc_aggregate(y1, src_pad, dst_pad, zeros_f, cpw).reshape(NC, NPAD, F)
    y2 = _tc_middle(agg1, y1, degs, W2, b1.reshape(1, F))
    agg2 = _sc_aggregate(y2, src_pad, dst_pad, zeros_f, cpw).reshape(NC, NPAD, F)
    out = _tc_last(agg2, y2, degs, b2.reshape(1, F))
    return out[:N_NODES]
